# baseline (speedup 1.0000x reference)
import jax
import jax.numpy as jnp
from jax import lax
from jax.experimental import pallas as pl
from jax.experimental.pallas import tpu as pltpu
from jax.experimental.pallas import tpu_sc as plsc
import functools

NC = 2
NS = 16
NW = NC * NS
L = 16

DIM = 32


def _bpr_body(nrows, v_users, v_items,
              u_hbm, i_hbm, j_hbm, ut_hbm, it_hbm,
              pos_hbm, neg_hbm,
              uidx_v, iidx_v, jidx_v, uoff_v, ioff_v, joff_v,
              uval_v, ival_v, jval_v, pos_v, neg_v,
              usem, isem, jsem):
    b_per_w = nrows * 128 // DIM
    ngroups = b_per_w // L
    wid = lax.axis_index("s") * NC + lax.axis_index("c")
    base = wid * b_per_w

    pltpu.sync_copy(u_hbm.at[pl.ds(base, b_per_w)], uidx_v)
    pltpu.sync_copy(i_hbm.at[pl.ds(base, b_per_w)], iidx_v)
    pltpu.sync_copy(j_hbm.at[pl.ds(base, b_per_w)], jidx_v)

    iota = lax.iota(jnp.int32, L)

    def off_group(g, _):
        uids = plsc.load_gather(uidx_v, [g * L + iota])
        iids = plsc.load_gather(iidx_v, [g * L + iota])
        jids = plsc.load_gather(jidx_v, [g * L + iota])
        for cc in range(4):
            for dl in range(8):
                d = cc * 8 + dl
                sl = pl.ds(dl * L, L)
                uoff_v[g * 4 + cc, sl] = uids + d * v_users
                ioff_v[g * 4 + cc, sl] = iids + d * v_items
                joff_v[g * 4 + cc, sl] = jids + d * v_items
        return 0

    lax.fori_loop(0, ngroups, off_group, 0)

    def fire(c, _):
        pltpu.async_copy(ut_hbm.at[uoff_v.at[c]], uval_v.at[c], usem)
        pltpu.async_copy(it_hbm.at[ioff_v.at[c]], ival_v.at[c], isem)
        pltpu.async_copy(it_hbm.at[joff_v.at[c]], jval_v.at[c], jsem)
        return 0

    lax.fori_loop(0, nrows, fire, 0)

    def drain(c, _):
        dummy = ut_hbm.at[pl.ds(0, 128)]
        pltpu.make_async_copy(dummy, uval_v.at[c], usem).wait()
        pltpu.make_async_copy(dummy, ival_v.at[c], isem).wait()
        pltpu.make_async_copy(dummy, jval_v.at[c], jsem).wait()
        return 0

    lax.fori_loop(0, nrows, drain, 0)

    def dot_group(g, _):
        pos = jnp.zeros((L,), jnp.float32)
        neg = jnp.zeros((L,), jnp.float32)
        for cc in range(4):
            for dl in range(8):
                sl = pl.ds(dl * L, L)
                ud = uval_v[g * 4 + cc, sl]
                pos = pos + ud * ival_v[g * 4 + cc, sl]
                neg = neg + ud * jval_v[g * 4 + cc, sl]
        pos_v[pl.ds(g * L, L)] = pos
        neg_v[pl.ds(g * L, L)] = neg
        return 0

    lax.fori_loop(0, ngroups, dot_group, 0)

    pltpu.sync_copy(pos_v, pos_hbm.at[pl.ds(base, b_per_w)])
    pltpu.sync_copy(neg_v, neg_hbm.at[pl.ds(base, b_per_w)])


def kernel(u, i, j, user_table, item_table):
    batch = u.shape[0]
    v_users = user_table.shape[0]
    v_items = item_table.shape[0]
    assert batch % (NW * 2 * L) == 0
    b_per_w = batch // NW
    nrows = b_per_w * DIM // 128

    ut1 = user_table.T.reshape(v_users * DIM)
    it1 = item_table.T.reshape(v_items * DIM)

    mesh = plsc.VectorSubcoreMesh(core_axis_name="c", subcore_axis_name="s",
                                  num_cores=NC, num_subcores=NS)
    f32 = jnp.float32
    run = pl.kernel(
        functools.partial(_bpr_body, nrows, v_users, v_items),
        out_type=(jax.ShapeDtypeStruct((batch,), f32),
                  jax.ShapeDtypeStruct((batch,), f32)),
        mesh=mesh,
        compiler_params=pltpu.CompilerParams(needs_layout_passes=False,
                                             use_tc_tiling_on_sc=False),
        scratch_types=[
            pltpu.VMEM((b_per_w,), jnp.int32),
            pltpu.VMEM((b_per_w,), jnp.int32),
            pltpu.VMEM((b_per_w,), jnp.int32),
            pltpu.VMEM((nrows, 128), jnp.int32),
            pltpu.VMEM((nrows, 128), jnp.int32),
            pltpu.VMEM((nrows, 128), jnp.int32),
            pltpu.VMEM((nrows, 128), f32),
            pltpu.VMEM((nrows, 128), f32),
            pltpu.VMEM((nrows, 128), f32),
            pltpu.VMEM((b_per_w,), f32),
            pltpu.VMEM((b_per_w,), f32),
            pltpu.SemaphoreType.DMA,
            pltpu.SemaphoreType.DMA,
            pltpu.SemaphoreType.DMA,
        ],
    )
    return run(u.astype(jnp.int32), i.astype(jnp.int32), j.astype(jnp.int32),
               ut1, it1)


# speedup vs baseline: 1.0023x; 1.0023x over previous
import jax
import jax.numpy as jnp
from jax import lax
from jax.experimental import pallas as pl
from jax.experimental.pallas import tpu as pltpu
from jax.experimental.pallas import tpu_sc as plsc
import functools

NC = 2
NS = 16
NW = NC * NS
L = 16

DIM = 32
PASS_IDS = 64
NGRP = PASS_IDS // L
ROWS_PER_PASS = PASS_IDS * DIM // 128


def _bpr_body(npass, hi_u, hi_i,
              u_hbm, i_hbm, j_hbm, ut_hbm, it_hbm,
              pos_hbm, neg_hbm,
              uidx_v, iidx_v, jidx_v, uoff_v, ioff_v, joff_v,
              uval_v, ival_v, jval_v, pos_v, neg_v,
              usem, isem, jsem):
    b_per_w = npass * PASS_IDS
    wid = lax.axis_index("s") * NC + lax.axis_index("c")
    base = wid * b_per_w

    pltpu.sync_copy(u_hbm.at[pl.ds(base, b_per_w)], uidx_v)
    pltpu.sync_copy(i_hbm.at[pl.ds(base, b_per_w)], iidx_v)
    pltpu.sync_copy(j_hbm.at[pl.ds(base, b_per_w)], jidx_v)

    iota = lax.iota(jnp.int32, L)

    def one_pass(p, _):
        for gg in range(NGRP):
            sl16 = pl.ds(p * PASS_IDS + gg * L, L)
            uh = plsc.load_gather(uidx_v, [p * PASS_IDS + gg * L + iota]) >> 4
            ih = plsc.load_gather(iidx_v, [p * PASS_IDS + gg * L + iota]) >> 4
            jh = plsc.load_gather(jidx_v, [p * PASS_IDS + gg * L + iota]) >> 4
            for d in range(DIM):
                sl = pl.ds((d % 2) * PASS_IDS + gg * L, L)
                uoff_v[d // 2, sl] = uh + d * hi_u
                ioff_v[d // 2, sl] = ih + d * hi_i
                joff_v[d // 2, sl] = jh + d * hi_i
        for rr in range(ROWS_PER_PASS):
            dsl = pl.ds(rr * 128, 128)
            pltpu.async_copy(ut_hbm.at[uoff_v.at[rr]], uval_v.at[dsl], usem)
            pltpu.async_copy(it_hbm.at[ioff_v.at[rr]], ival_v.at[dsl], isem)
            pltpu.async_copy(it_hbm.at[joff_v.at[rr]], jval_v.at[dsl], jsem)
        for rr in range(ROWS_PER_PASS):
            dummy = ut_hbm.at[pl.ds(0, 128)]
            dsl = pl.ds(rr * 128, 128)
            pltpu.make_async_copy(dummy, uval_v.at[dsl], usem).wait()
            pltpu.make_async_copy(dummy, ival_v.at[dsl], isem).wait()
            pltpu.make_async_copy(dummy, jval_v.at[dsl], jsem).wait()
        for gg in range(NGRP):
            ids_u = plsc.load_gather(uidx_v, [p * PASS_IDS + gg * L + iota])
            ids_i = plsc.load_gather(iidx_v, [p * PASS_IDS + gg * L + iota])
            ids_j = plsc.load_gather(jidx_v, [p * PASS_IDS + gg * L + iota])
            lu = ids_u & 15
            li = ids_i & 15
            lj = ids_j & 15
            pos = jnp.zeros((L,), jnp.float32)
            neg = jnp.zeros((L,), jnp.float32)
            for d in range(DIM):
                q = d * PASS_IDS + gg * L + iota
                ud = plsc.load_gather(uval_v, [q, lu])
                vd = plsc.load_gather(ival_v, [q, li])
                wd = plsc.load_gather(jval_v, [q, lj])
                pos = pos + ud * vd
                neg = neg + ud * wd
            osl = pl.ds(p * PASS_IDS + gg * L, L)
            pos_v[osl] = pos
            neg_v[osl] = neg
        return 0

    lax.fori_loop(0, npass, one_pass, 0)

    pltpu.sync_copy(pos_v, pos_hbm.at[pl.ds(base, b_per_w)])
    pltpu.sync_copy(neg_v, neg_hbm.at[pl.ds(base, b_per_w)])


def kernel(u, i, j, user_table, item_table):
    batch = u.shape[0]
    v_users = user_table.shape[0]
    v_items = item_table.shape[0]
    assert v_users % 16 == 0 and v_items % 16 == 0
    assert batch % (NW * PASS_IDS) == 0
    b_per_w = batch // NW
    npass = b_per_w // PASS_IDS

    ut2 = user_table.T.reshape(v_users * DIM // 16, 16)
    it2 = item_table.T.reshape(v_items * DIM // 16, 16)

    mesh = plsc.VectorSubcoreMesh(core_axis_name="c", subcore_axis_name="s",
                                  num_cores=NC, num_subcores=NS)
    f32 = jnp.float32
    nv = PASS_IDS * DIM
    run = pl.kernel(
        functools.partial(_bpr_body, npass, v_users // 16, v_items // 16),
        out_type=(jax.ShapeDtypeStruct((batch,), f32),
                  jax.ShapeDtypeStruct((batch,), f32)),
        mesh=mesh,
        compiler_params=pltpu.CompilerParams(needs_layout_passes=False,
                                             use_tc_tiling_on_sc=False),
        scratch_types=[
            pltpu.VMEM((b_per_w,), jnp.int32),
            pltpu.VMEM((b_per_w,), jnp.int32),
            pltpu.VMEM((b_per_w,), jnp.int32),
            pltpu.VMEM((ROWS_PER_PASS, 128), jnp.int32),
            pltpu.VMEM((ROWS_PER_PASS, 128), jnp.int32),
            pltpu.VMEM((ROWS_PER_PASS, 128), jnp.int32),
            pltpu.VMEM((nv, 16), f32),
            pltpu.VMEM((nv, 16), f32),
            pltpu.VMEM((nv, 16), f32),
            pltpu.VMEM((b_per_w,), f32),
            pltpu.VMEM((b_per_w,), f32),
            pltpu.SemaphoreType.DMA,
            pltpu.SemaphoreType.DMA,
            pltpu.SemaphoreType.DMA,
        ],
    )
    return run(u.astype(jnp.int32), i.astype(jnp.int32), j.astype(jnp.int32),
               ut2, it2)


# speedup vs baseline: 5.5861x; 5.5732x over previous
import jax
import jax.numpy as jnp
from jax import lax
from jax.experimental import pallas as pl
from jax.experimental.pallas import tpu as pltpu
from jax.experimental.pallas import tpu_sc as plsc
import functools

NC = 2
NS = 16
NW = NC * NS
L = 16

DIM = 32
CHUNK = 128


def _bpr_body(nchunks, u_hbm, i_hbm, j_hbm, ut_hbm, it_hbm,
              pos_hbm, neg_hbm,
              uidx_v, iidx_v, jidx_v, urows_v, irows_v, jrows_v,
              pos_v, neg_v, sem):
    b_per_w = nchunks * CHUNK
    wid = lax.axis_index("s") * NC + lax.axis_index("c")
    base = wid * b_per_w

    for c in range(nchunks):
        pltpu.sync_copy(u_hbm.at[pl.ds(base + c * CHUNK, CHUNK)], uidx_v.at[c])
        pltpu.sync_copy(i_hbm.at[pl.ds(base + c * CHUNK, CHUNK)], iidx_v.at[c])
        pltpu.sync_copy(j_hbm.at[pl.ds(base + c * CHUNK, CHUNK)], jidx_v.at[c])

    copies = []
    for c in range(nchunks):
        sl = pl.ds(c * CHUNK, CHUNK)
        copies.append(pltpu.async_copy(ut_hbm.at[uidx_v.at[c]], urows_v.at[sl], sem))
        copies.append(pltpu.async_copy(it_hbm.at[iidx_v.at[c]], irows_v.at[sl], sem))
        copies.append(pltpu.async_copy(it_hbm.at[jidx_v.at[c]], jrows_v.at[sl], sem))
    for cp in copies:
        cp.wait()

    iota = lax.iota(jnp.int32, L)

    def group_body(g, _):
        ridx = g * L + iota
        pos = jnp.zeros((L,), jnp.float32)
        neg = jnp.zeros((L,), jnp.float32)
        for d in range(DIM):
            didx = jnp.full((L,), d, jnp.int32)
            ud = plsc.load_gather(urows_v, [ridx, didx])
            vd = plsc.load_gather(irows_v, [ridx, didx])
            wd = plsc.load_gather(jrows_v, [ridx, didx])
            pos = pos + ud * vd
            neg = neg + ud * wd
        pos_v[pl.ds(g * L, L)] = pos
        neg_v[pl.ds(g * L, L)] = neg
        return 0

    lax.fori_loop(0, b_per_w // L, group_body, 0)

    pltpu.sync_copy(pos_v, pos_hbm.at[pl.ds(base, b_per_w)])
    pltpu.sync_copy(neg_v, neg_hbm.at[pl.ds(base, b_per_w)])


def kernel(u, i, j, user_table, item_table):
    batch = u.shape[0]
    assert batch % (NW * CHUNK) == 0
    nchunks = batch // (NW * CHUNK)
    b_per_w = nchunks * CHUNK

    mesh = plsc.VectorSubcoreMesh(core_axis_name="c", subcore_axis_name="s",
                                  num_cores=NC, num_subcores=NS)
    f32 = jnp.float32
    run = pl.kernel(
        functools.partial(_bpr_body, nchunks),
        out_type=(jax.ShapeDtypeStruct((batch,), f32),
                  jax.ShapeDtypeStruct((batch,), f32)),
        mesh=mesh,
        compiler_params=pltpu.CompilerParams(needs_layout_passes=False,
                                             use_tc_tiling_on_sc=False),
        scratch_types=[
            pltpu.VMEM((nchunks, CHUNK), jnp.int32),
            pltpu.VMEM((nchunks, CHUNK), jnp.int32),
            pltpu.VMEM((nchunks, CHUNK), jnp.int32),
            pltpu.VMEM((b_per_w, DIM), f32),
            pltpu.VMEM((b_per_w, DIM), f32),
            pltpu.VMEM((b_per_w, DIM), f32),
            pltpu.VMEM((b_per_w,), f32),
            pltpu.VMEM((b_per_w,), f32),
            pltpu.SemaphoreType.DMA,
        ],
    )
    return run(u.astype(jnp.int32), i.astype(jnp.int32), j.astype(jnp.int32),
               user_table, item_table)
